# baseline (speedup 1.0000x reference)
import functools

import jax
import jax.numpy as jnp
from jax.experimental import pallas as pl
from jax.experimental.pallas import tpu as pltpu


def _feature_kernel(v_ref, w_ref, x_ref):
    x_ref[...] = jnp.dot(v_ref[...], w_ref[...],
                         preferred_element_type=jnp.float32)


def _outer_kernel(adj_hbm, x_ref, bias_ref, out_ref, *, n, bm, out_f):
    def body(adj_blk, out_blk):
        adj = adj_blk[...]
        xs = x_ref[...]
        a1 = (adj == 1).astype(jnp.float32)
        a2 = (adj == 2).astype(jnp.float32)
        a3 = (adj == 3).astype(jnp.float32)
        acc = jnp.dot(a1, xs[:, :out_f], preferred_element_type=jnp.float32)
        acc += jnp.dot(a2, xs[:, out_f:2 * out_f],
                       preferred_element_type=jnp.float32)
        acc += jnp.dot(a3, xs[:, 2 * out_f:],
                       preferred_element_type=jnp.float32)
        out_blk[...] = acc + bias_ref[...]

    pltpu.emit_pipeline(
        body,
        grid=(n // bm,),
        in_specs=[
            pl.BlockSpec((bm, n), lambda i: (i, 0),
                         pipeline_mode=pl.Buffered(buffer_count=4)),
        ],
        out_specs=[pl.BlockSpec((bm, out_f), lambda i: (i, 0))],
    )(adj_hbm, out_ref)


def kernel(V, adj, w1, w2, w3, bias):
    n, in_f = V.shape
    out_f = w1.shape[1]
    w = jnp.concatenate([w1, w2, w3], axis=1)

    bm_x = 1024
    x = pl.pallas_call(
        _feature_kernel,
        grid=(n // bm_x,),
        in_specs=[
            pl.BlockSpec((bm_x, in_f), lambda i: (i, 0)),
            pl.BlockSpec((in_f, 3 * out_f), lambda i: (0, 0)),
        ],
        out_specs=pl.BlockSpec((bm_x, 3 * out_f), lambda i: (i, 0)),
        out_shape=jax.ShapeDtypeStruct((n, 3 * out_f), jnp.float32),
    )(V, w)

    bm = 256
    body = functools.partial(_outer_kernel, n=n, bm=bm, out_f=out_f)
    out = pl.pallas_call(
        body,
        in_specs=[
            pl.BlockSpec(memory_space=pl.ANY),
            pl.BlockSpec(memory_space=pltpu.VMEM),
            pl.BlockSpec(memory_space=pltpu.VMEM),
        ],
        out_specs=pl.BlockSpec(memory_space=pl.ANY),
        out_shape=jax.ShapeDtypeStruct((n, out_f), jnp.float32),
    )(adj, x, bias.reshape(1, out_f))
    return out


# speedup vs baseline: 1.0352x; 1.0352x over previous
import functools

import jax
import jax.numpy as jnp
from jax.experimental import pallas as pl
from jax.experimental.pallas import tpu as pltpu


def _feature_kernel(v_ref, w_ref, y_ref):
    y_ref[...] = jnp.dot(v_ref[...], w_ref[...],
                         preferred_element_type=jnp.float32)


def _spmm_kernel(adj_ref, y_ref, bias_ref, out_ref, *, out_f):
    adj = adj_ref[...]
    ys = y_ref[...]
    a1 = adj.astype(jnp.bfloat16)
    a2 = a1 * a1
    a3 = a2 * a1
    dn = (((1,), (0,)), ((), ()))
    acc = jax.lax.dot_general(a1, ys[:, :out_f], dn,
                              preferred_element_type=jnp.float32)
    acc += jax.lax.dot_general(a2, ys[:, out_f:2 * out_f], dn,
                               preferred_element_type=jnp.float32)
    acc += jax.lax.dot_general(a3, ys[:, 2 * out_f:], dn,
                               preferred_element_type=jnp.float32)
    out_ref[...] = acc + bias_ref[...]


def kernel(V, adj, w1, w2, w3, bias):
    n, in_f = V.shape
    out_f = w1.shape[1]
    u1 = 3.0 * w1 - 1.5 * w2 + w3 / 3.0
    u2 = -2.5 * w1 + 2.0 * w2 - 0.5 * w3
    u3 = 0.5 * w1 - 0.5 * w2 + w3 / 6.0
    w = jnp.concatenate([u1, u2, u3], axis=1)

    bm_x = 1024
    y = pl.pallas_call(
        _feature_kernel,
        grid=(n // bm_x,),
        in_specs=[
            pl.BlockSpec((bm_x, in_f), lambda i: (i, 0)),
            pl.BlockSpec((in_f, 3 * out_f), lambda i: (0, 0)),
        ],
        out_specs=pl.BlockSpec((bm_x, 3 * out_f), lambda i: (i, 0)),
        out_shape=jax.ShapeDtypeStruct((n, 3 * out_f), jnp.float32),
    )(V, w)

    bm = 256
    body = functools.partial(_spmm_kernel, out_f=out_f)
    out = pl.pallas_call(
        body,
        grid=(n // bm,),
        in_specs=[
            pl.BlockSpec((bm, n), lambda i: (i, 0)),
            pl.BlockSpec((n, 3 * out_f), lambda i: (0, 0)),
            pl.BlockSpec((1, out_f), lambda i: (0, 0)),
        ],
        out_specs=pl.BlockSpec((bm, out_f), lambda i: (i, 0)),
        out_shape=jax.ShapeDtypeStruct((n, out_f), jnp.float32),
        compiler_params=pltpu.CompilerParams(
            dimension_semantics=("arbitrary",),
        ),
    )(adj, y, bias.reshape(1, out_f))
    return out


# speedup vs baseline: 1.0561x; 1.0202x over previous
import functools

import jax
import jax.numpy as jnp
from jax.experimental import pallas as pl
from jax.experimental.pallas import tpu as pltpu


def _feature_kernel(v_ref, w_ref, x_ref):
    x_ref[...] = jnp.dot(v_ref[...], w_ref[...],
                         preferred_element_type=jnp.float32)


def _spmm_kernel(adj_ref, x_ref, bias_ref, out_ref, *, out_f):
    adj = adj_ref[...]
    xs = x_ref[...]
    a1 = (adj == 1).astype(jnp.float32)
    a2 = (adj == 2).astype(jnp.float32)
    a3 = (adj == 3).astype(jnp.float32)
    acc = jnp.dot(a1, xs[:, :out_f], preferred_element_type=jnp.float32)
    acc += jnp.dot(a2, xs[:, out_f:2 * out_f],
                   preferred_element_type=jnp.float32)
    acc += jnp.dot(a3, xs[:, 2 * out_f:],
                   preferred_element_type=jnp.float32)
    out_ref[...] = acc + bias_ref[...]


def kernel(V, adj, w1, w2, w3, bias):
    n, in_f = V.shape
    out_f = w1.shape[1]
    w = jnp.concatenate([w1, w2, w3], axis=1)

    bm_x = 1024
    x = pl.pallas_call(
        _feature_kernel,
        grid=(n // bm_x,),
        in_specs=[
            pl.BlockSpec((bm_x, in_f), lambda i: (i, 0)),
            pl.BlockSpec((in_f, 3 * out_f), lambda i: (0, 0)),
        ],
        out_specs=pl.BlockSpec((bm_x, 3 * out_f), lambda i: (i, 0)),
        out_shape=jax.ShapeDtypeStruct((n, 3 * out_f), jnp.float32),
    )(V, w)

    bm = 512
    body = functools.partial(_spmm_kernel, out_f=out_f)
    out = pl.pallas_call(
        body,
        grid=(n // bm,),
        in_specs=[
            pl.BlockSpec((bm, n), lambda i: (i, 0)),
            pl.BlockSpec((n, 3 * out_f), lambda i: (0, 0)),
            pl.BlockSpec((1, out_f), lambda i: (0, 0)),
        ],
        out_specs=pl.BlockSpec((bm, out_f), lambda i: (i, 0)),
        out_shape=jax.ShapeDtypeStruct((n, out_f), jnp.float32),
        compiler_params=pltpu.CompilerParams(
            dimension_semantics=("parallel",),
        ),
    )(adj, x, bias.reshape(1, out_f))
    return out
